# Initial kernel scaffold; baseline (speedup 1.0000x reference)
import functools

import jax
import jax.numpy as jnp
from jax import lax
from jax.experimental import pallas as pl
from jax.experimental.pallas import tpu as pltpu
from jax.experimental.pallas import tpu_sc as plsc

_N_DATA = 1_000_000
_RHO = 0.99
_ALPHA = 0.5
_EPS = 1e-6
_B = 4096
_NC = 2
_NS = 16
_NW = _NC * _NS
_WIN = 31264
_NVEC = _B // 16


def _dense_body(feats_ref, c_ref, r_ref, att_ref):
    fa = feats_ref[0:_B, :]
    fb = feats_ref[_B:2 * _B, :]
    fa_roll = jnp.concatenate([fa[1:], fa[:1]], axis=0)
    fb_roll = jnp.concatenate([fb[1:], fb[:1]], axis=0)

    def d2(x):
        return jnp.sum((x * x).reshape(32, 128, 128), axis=2)

    da2 = d2(fa - fb + _EPS)
    db2 = d2(fb - fa + _EPS)
    dra2 = d2(fa - fb_roll + _EPS)
    drb2 = d2(fb - fa_roll + _EPS)
    qa = 1.0 / (1.0 + da2)
    qb = 1.0 / (1.0 + db2)
    qra = 1.0 / (1.0 + dra2)
    qrb = 1.0 / (1.0 + drb2)
    att_ref[0, 0] = (jnp.sum(-jnp.log(qa)) + jnp.sum(-jnp.log(qb))) / (2.0 * _B)
    npow2 = jnp.float32(_N_DATA) ** 2
    ema = (1.0 - _RHO) * npow2
    xi_a = _ALPHA * qa + (1.0 - _ALPHA) * qra
    xi_b = _ALPHA * qb + (1.0 - _ALPHA) * qrb
    c_ref[...] = (ema * xi_a + ema * xi_b) * 0.5
    r_ref[...] = qra + qrb


_dense_call = pl.pallas_call(
    _dense_body,
    out_shape=(
        jax.ShapeDtypeStruct((32, 128), jnp.float32),
        jax.ShapeDtypeStruct((32, 128), jnp.float32),
        jax.ShapeDtypeStruct((1, 1), jnp.float32),
    ),
)


_sc_mesh = plsc.VectorSubcoreMesh(
    core_axis_name="c", subcore_axis_name="s", num_cores=_NC, num_subcores=_NS
)


@functools.partial(
    pl.kernel,
    out_type=(
        jax.ShapeDtypeStruct((_N_DATA,), jnp.float32),
        jax.ShapeDtypeStruct((_NW * 16,), jnp.float32),
    ),
    mesh=_sc_mesh,
    scratch_types=[
        pltpu.VMEM((_WIN,), jnp.float32),
        pltpu.VMEM((_B,), jnp.int32),
        pltpu.VMEM((_B,), jnp.float32),
        pltpu.VMEM((_B,), jnp.float32),
        pltpu.VMEM((_B,), jnp.float32),
        pltpu.VMEM((16,), jnp.float32),
    ],
)
def _sc_update(s_inv_hbm, idx_hbm, c_hbm, r_hbm, out_hbm, parts_hbm,
               win_v, idx_v, c_v, r_v, scur_v, part_v):
    wid = lax.axis_index("s") * _NC + lax.axis_index("c")
    base = jnp.minimum(wid * _WIN, _N_DATA - _WIN)
    obase = wid * _WIN
    oend = jnp.minimum(obase + _WIN, _N_DATA)

    pltpu.sync_copy(s_inv_hbm.at[pl.ds(base, _WIN)], win_v)
    pltpu.sync_copy(idx_hbm, idx_v)
    pltpu.sync_copy(c_hbm, c_v)
    pltpu.sync_copy(r_hbm, r_v)

    def gather_body(i, acc):
        sl = pl.ds(i * 16, 16)
        iv = idx_v[sl]
        off = iv - base
        win = (off >= 0) & (off < _WIN)
        offc = jnp.clip(off, 0, _WIN - 1)
        s_cur = plsc.load_gather(win_v, [offc], mask=win)
        scur_v[sl] = s_cur
        own = (iv >= obase) & (iv < oend)
        return acc + jnp.where(own, r_v[sl] / s_cur, 0.0)

    acc = lax.fori_loop(0, _NVEC, gather_body, jnp.zeros((16,), jnp.float32))

    def scatter_body(i, carry):
        sl = pl.ds(i * 16, 16)
        iv = idx_v[sl]
        off = iv - base
        win = (off >= 0) & (off < _WIN)
        offc = jnp.clip(off, 0, _WIN - 1)
        _, last = plsc.scan_count(iv)
        v = _RHO * scur_v[sl] + c_v[sl]
        plsc.store_scatter(win_v, [offc], v, mask=win & last)
        return carry

    lax.fori_loop(0, _NVEC, scatter_body, 0)

    pltpu.sync_copy(win_v, out_hbm.at[pl.ds(base, _WIN)])
    part_v[...] = jnp.broadcast_to(jnp.sum(acc), (16,))
    pltpu.sync_copy(part_v, parts_hbm.at[pl.ds(wid * 16, 16)])


def kernel(feats, feats_idx, s_inv):
    c2, r2, att = _dense_call(feats)
    new_s_inv, parts = _sc_update(
        s_inv, feats_idx, c2.reshape(_B), r2.reshape(_B)
    )
    npow2 = jnp.float32(_N_DATA) ** 2
    rep = jnp.sum(parts.reshape(_NW, 16)[:, 0]) * (npow2 / jnp.float32(2 * _B))
    loss = att[0, 0] + rep
    return loss, new_s_inv


# speedup vs baseline: 1.6536x; 1.6536x over previous
import functools

import jax
import jax.numpy as jnp
from jax import lax
from jax.experimental import pallas as pl
from jax.experimental.pallas import tpu as pltpu
from jax.experimental.pallas import tpu_sc as plsc

_N_DATA = 1_000_000
_RHO = 0.99
_ALPHA = 0.5
_EPS = 1e-6
_B = 4096
_NC = 2
_NS = 16
_NW = _NC * _NS
_WIN = 31264
_NVEC = _B // 16


def _dense_body(feats_ref, c_ref, r_ref, att_ref):
    fa = feats_ref[0:_B, :]
    fb = feats_ref[_B:2 * _B, :]
    fa_roll = jnp.concatenate([fa[1:], fa[:1]], axis=0)
    fb_roll = jnp.concatenate([fb[1:], fb[:1]], axis=0)

    def d2(x):
        return jnp.sum((x * x).reshape(32, 128, 128), axis=2)

    da2 = d2(fa - fb + _EPS)
    db2 = d2(fb - fa + _EPS)
    dra2 = d2(fa - fb_roll + _EPS)
    drb2 = d2(fb - fa_roll + _EPS)
    qa = 1.0 / (1.0 + da2)
    qb = 1.0 / (1.0 + db2)
    qra = 1.0 / (1.0 + dra2)
    qrb = 1.0 / (1.0 + drb2)
    att = (jnp.sum(-jnp.log(qa)) + jnp.sum(-jnp.log(qb))) / (2.0 * _B)
    att_ref[...] = jnp.broadcast_to(att, (1, 1))
    npow2 = jnp.float32(_N_DATA) ** 2
    ema = (1.0 - _RHO) * npow2
    xi_a = _ALPHA * qa + (1.0 - _ALPHA) * qra
    xi_b = _ALPHA * qb + (1.0 - _ALPHA) * qrb
    c_ref[...] = (ema * xi_a + ema * xi_b) * 0.5
    r_ref[...] = qra + qrb


_dense_call = pl.pallas_call(
    _dense_body,
    out_shape=(
        jax.ShapeDtypeStruct((32, 128), jnp.float32),
        jax.ShapeDtypeStruct((32, 128), jnp.float32),
        jax.ShapeDtypeStruct((1, 1), jnp.float32),
    ),
)


_sc_mesh = plsc.VectorSubcoreMesh(
    core_axis_name="c", subcore_axis_name="s", num_cores=_NC, num_subcores=_NS
)


@functools.partial(
    pl.kernel,
    out_type=(
        jax.ShapeDtypeStruct((_N_DATA,), jnp.float32),
        jax.ShapeDtypeStruct((_NW * 16,), jnp.float32),
    ),
    mesh=_sc_mesh,
    compiler_params=pltpu.CompilerParams(needs_layout_passes=False),
    scratch_types=[
        pltpu.VMEM((_WIN,), jnp.float32),
        pltpu.VMEM((_B,), jnp.int32),
        pltpu.VMEM((_B,), jnp.float32),
        pltpu.VMEM((_B,), jnp.float32),
        pltpu.VMEM((_B,), jnp.float32),
        pltpu.VMEM((16,), jnp.float32),
    ],
)
def _sc_update(s_inv_hbm, idx_hbm, c_hbm, r_hbm, out_hbm, parts_hbm,
               win_v, idx_v, c_v, r_v, scur_v, part_v):
    wid = lax.axis_index("s") * _NC + lax.axis_index("c")
    base = jnp.minimum(wid * _WIN, _N_DATA - _WIN)
    obase = wid * _WIN
    oend = jnp.minimum(obase + _WIN, _N_DATA)

    pltpu.sync_copy(s_inv_hbm.at[pl.ds(base, _WIN)], win_v)
    pltpu.sync_copy(idx_hbm, idx_v)
    pltpu.sync_copy(c_hbm, c_v)
    pltpu.sync_copy(r_hbm, r_v)

    def gather_body(i, acc):
        sl = pl.ds(i * 16, 16)
        iv = idx_v[sl]
        off = iv - base
        win = (off >= 0) & (off < _WIN)
        offc = jnp.clip(off, 0, _WIN - 1)
        s_cur = plsc.load_gather(win_v, [offc], mask=win)
        scur_v[sl] = s_cur
        own = (iv >= obase) & (iv < oend)
        return acc + jnp.where(own, r_v[sl] / s_cur, 0.0)

    acc = lax.fori_loop(0, _NVEC, gather_body, jnp.zeros((16,), jnp.float32))

    def scatter_body(i, carry):
        sl = pl.ds(i * 16, 16)
        iv = idx_v[sl]
        off = iv - base
        win = (off >= 0) & (off < _WIN)
        offc = jnp.clip(off, 0, _WIN - 1)
        _, last = plsc.scan_count(iv)
        v = _RHO * scur_v[sl] + c_v[sl]
        plsc.store_scatter(win_v, [offc], v, mask=win & last)
        return carry

    lax.fori_loop(0, _NVEC, scatter_body, 0)

    pltpu.sync_copy(win_v, out_hbm.at[pl.ds(base, _WIN)])
    part_v[...] = jnp.broadcast_to(jnp.sum(acc), (16,))
    pltpu.sync_copy(part_v, parts_hbm.at[pl.ds(wid * 16, 16)])


def kernel(feats, feats_idx, s_inv):
    c2, r2, att = _dense_call(feats)
    new_s_inv, parts = _sc_update(
        s_inv, feats_idx, c2.reshape(_B), r2.reshape(_B)
    )
    npow2 = jnp.float32(_N_DATA) ** 2
    rep = jnp.sum(parts.reshape(_NW, 16)[:, 0]) * (npow2 / jnp.float32(2 * _B))
    loss = att[0, 0] + rep
    return loss, new_s_inv
